# Initial kernel scaffold; baseline (speedup 1.0000x reference)
import functools

import jax
import jax.numpy as jnp
from jax import lax
from jax.experimental import pallas as pl
from jax.experimental.pallas import tpu as pltpu
from jax.experimental.pallas import tpu_sc as plsc

_D = 64
_NC, _NS = 2, 16
_NW = _NC * _NS
_CHUNK = 128


@functools.cache
def _make_lookup(B):
    b_per_w = B // _NW
    n_chunks = b_per_w // _CHUNK

    def body(idx_hbm, table_hbm, out_hbm, idx_v, rows_v, sem):
        wid = lax.axis_index("s") * _NC + lax.axis_index("c")
        base = wid * b_per_w

        def step(i, carry):
            off = base + i * _CHUNK
            pltpu.sync_copy(idx_hbm.at[pl.ds(off, _CHUNK)], idx_v)
            pltpu.async_copy(table_hbm.at[idx_v], rows_v, sem).wait()
            pltpu.sync_copy(rows_v, out_hbm.at[pl.ds(off, _CHUNK)])
            return carry

        lax.fori_loop(0, n_chunks, step, 0)

    return pl.kernel(
        body,
        mesh=plsc.VectorSubcoreMesh(core_axis_name="c", subcore_axis_name="s"),
        out_type=jax.ShapeDtypeStruct((B, _D), jnp.float32),
        scratch_types=[
            pltpu.VMEM((_CHUNK,), jnp.int32),
            pltpu.VMEM((_CHUNK, _D), jnp.float32),
            pltpu.SemaphoreType.DMA,
        ],
    )


def kernel(token_ids, weight):
    B = token_ids.size
    flat = token_ids.reshape(-1).astype(jnp.int32)
    out = _make_lookup(B)(flat, weight)
    return out.reshape(*token_ids.shape, weight.shape[1])


# speedup vs baseline: 1.5723x; 1.5723x over previous
import functools

import jax
import jax.numpy as jnp
from jax import lax
from jax.experimental import pallas as pl
from jax.experimental.pallas import tpu as pltpu
from jax.experimental.pallas import tpu_sc as plsc

_D = 64
_NC, _NS = 2, 16
_NW = _NC * _NS
_CHUNK = 128


@functools.cache
def _make_lookup(B):
    b_per_w = B // _NW
    n_chunks = b_per_w // _CHUNK

    def body(idx_hbm, table_hbm, out_hbm, idx_v, rows_v, sem):
        wid = lax.axis_index("s") * _NC + lax.axis_index("c")
        base = wid * b_per_w

        def step(i, carry):
            off = base + i * _CHUNK
            pltpu.sync_copy(idx_hbm.at[pl.ds(off, _CHUNK)], idx_v)
            pltpu.async_copy(table_hbm.at[idx_v], rows_v, sem).wait()
            pltpu.sync_copy(rows_v, out_hbm.at[pl.ds(off, _CHUNK)])
            return carry

        lax.fori_loop(0, n_chunks, step, 0)

    return pl.kernel(
        body,
        mesh=plsc.VectorSubcoreMesh(core_axis_name="c", subcore_axis_name="s"),
        compiler_params=pltpu.CompilerParams(use_tc_tiling_on_sc=False),
        out_type=jax.ShapeDtypeStruct((B, _D), jnp.float32),
        scratch_types=[
            pltpu.VMEM((_CHUNK,), jnp.int32),
            pltpu.VMEM((_CHUNK, _D), jnp.float32),
            pltpu.SemaphoreType.DMA,
        ],
    )


def kernel(token_ids, weight):
    B = token_ids.size
    flat = token_ids.reshape(-1).astype(jnp.int32)
    out = _make_lookup(B)(flat, weight)
    return out.reshape(*token_ids.shape, weight.shape[1])


# speedup vs baseline: 1.8738x; 1.1918x over previous
import functools

import jax
import jax.numpy as jnp
from jax import lax
from jax.experimental import pallas as pl
from jax.experimental.pallas import tpu as pltpu
from jax.experimental.pallas import tpu_sc as plsc

_D = 64
_NC, _NS = 2, 16
_NW = _NC * _NS
_CHUNK = 256
_NBUF = 4


@functools.cache
def _make_lookup(B):
    b_per_w = B // _NW
    n = b_per_w // _CHUNK
    assert B % (_NW * _CHUNK) == 0
    assert n > 2 * _NBUF and (n - _NBUF) % _NBUF == 0

    def body(idx_hbm, table_hbm, out_hbm, idx_v, rows, gsem, wsem):
        wid = lax.axis_index("s") * _NC + lax.axis_index("c")
        base = wid * b_per_w

        def g_start(i, b):
            pltpu.async_copy(
                table_hbm.at[idx_v.at[pl.ds(i * _CHUNK, _CHUNK)]],
                rows.at[b], gsem.at[b])

        def g_wait(i, b):
            pltpu.make_async_copy(
                table_hbm.at[idx_v.at[pl.ds(i * _CHUNK, _CHUNK)]],
                rows.at[b], gsem.at[b]).wait()

        def w_start(i, b):
            pltpu.async_copy(
                rows.at[b], out_hbm.at[pl.ds(base + i * _CHUNK, _CHUNK)],
                wsem.at[b])

        def w_wait(i, b):
            pltpu.make_async_copy(
                rows.at[b], out_hbm.at[pl.ds(base + i * _CHUNK, _CHUNK)],
                wsem.at[b]).wait()

        pltpu.sync_copy(idx_hbm.at[pl.ds(base, b_per_w)], idx_v)

        for b in range(_NBUF - 1):
            g_start(b, b)
        g_start(_NBUF - 1, _NBUF - 1)
        g_wait(0, 0)
        w_start(0, 0)

        def blk(k, carry):
            for b in range(_NBUF):
                i = 1 + k * _NBUF + b
                buf = (1 + b) % _NBUF
                prev = (buf - 1) % _NBUF
                w_wait(i - 1, prev)
                g_start(i + _NBUF - 1, prev)
                g_wait(i, buf)
                w_start(i, buf)
            return carry

        lax.fori_loop(0, (n - _NBUF) // _NBUF, blk, 0)

        for i in range(n - _NBUF + 1, n):
            buf = i % _NBUF
            w_wait(i - 1, (buf - 1) % _NBUF)
            g_wait(i, buf)
            w_start(i, buf)
        w_wait(n - 1, (n - 1) % _NBUF)

    return pl.kernel(
        body,
        mesh=plsc.VectorSubcoreMesh(core_axis_name="c", subcore_axis_name="s"),
        compiler_params=pltpu.CompilerParams(use_tc_tiling_on_sc=False),
        out_type=jax.ShapeDtypeStruct((B, _D), jnp.float32),
        scratch_types=[
            pltpu.VMEM((B // _NW,), jnp.int32),
            pltpu.VMEM((_NBUF, _CHUNK, _D), jnp.float32),
            pltpu.SemaphoreType.DMA((_NBUF,)),
            pltpu.SemaphoreType.DMA((_NBUF,)),
        ],
    )


def kernel(token_ids, weight):
    B = token_ids.size
    flat = token_ids.reshape(-1).astype(jnp.int32)
    out = _make_lookup(B)(flat, weight)
    return out.reshape(*token_ids.shape, weight.shape[1])
